# baseline (speedup 1.0000x reference)
import functools

import jax
import jax.numpy as jnp
from jax.experimental import pallas as pl
from jax.experimental.pallas import tpu as pltpu

_NCHUNK = 8


def _body(feats_ref, tgt_ref, queue_ref, qlab_ref, out_ref, lab_ref,
          fvmem, ftvmem, copy_sems, lab_sem, tgt_sem, f_in_sem, f_out_sem,
          *, bsz, dim, k):
    rest = k - bsz
    cw = rest // _NCHUNK

    copies = []
    for i in range(_NCHUNK):
        lo = bsz + i * cw
        w = cw if i < _NCHUNK - 1 else rest - (_NCHUNK - 1) * cw
        cp = pltpu.make_async_copy(
            queue_ref.at[:, pl.ds(lo, w)],
            out_ref.at[:, pl.ds(lo, w)],
            copy_sems.at[i],
        )
        cp.start()
        copies.append(cp)

    lcp = pltpu.make_async_copy(
        qlab_ref.at[:, pl.ds(bsz, rest)], lab_ref.at[:, pl.ds(bsz, rest)],
        lab_sem)
    lcp.start()
    tcp = pltpu.make_async_copy(tgt_ref, lab_ref.at[:, pl.ds(0, bsz)], tgt_sem)
    tcp.start()

    fin = pltpu.make_async_copy(feats_ref, fvmem, f_in_sem)
    fin.start()
    fin.wait()
    ftvmem[...] = fvmem[...].T
    fout = pltpu.make_async_copy(ftvmem, out_ref.at[:, pl.ds(0, bsz)],
                                 f_out_sem)
    fout.start()
    fout.wait()

    for cp in copies:
        cp.wait()
    lcp.wait()
    tcp.wait()


def kernel(feats, targets, queue, queue_label):
    bsz, dim = feats.shape
    k = queue.shape[1]
    targets2d = targets.reshape(1, bsz)

    body = functools.partial(_body, bsz=bsz, dim=dim, k=k)
    hbm = pl.BlockSpec(memory_space=pltpu.MemorySpace.HBM)

    queue_new, label_new = pl.pallas_call(
        body,
        in_specs=[hbm, hbm, hbm, hbm],
        out_specs=[hbm, hbm],
        out_shape=[
            jax.ShapeDtypeStruct((dim, k), queue.dtype),
            jax.ShapeDtypeStruct((1, k), queue_label.dtype),
        ],
        scratch_shapes=[
            pltpu.VMEM((bsz, dim), feats.dtype),
            pltpu.VMEM((dim, bsz), feats.dtype),
            pltpu.SemaphoreType.DMA((_NCHUNK,)),
            pltpu.SemaphoreType.DMA,
            pltpu.SemaphoreType.DMA,
            pltpu.SemaphoreType.DMA,
            pltpu.SemaphoreType.DMA,
        ],
    )(feats, targets2d, queue, queue_label)

    new_ptr = jnp.full((1,), (0 + bsz) % k, dtype=jnp.int32)
    return queue_new, label_new, new_ptr


# speedup vs baseline: 43.5887x; 43.5887x over previous
import functools

import jax
import jax.numpy as jnp
from jax.experimental import pallas as pl

_BW = 512


def _body(feats_ref, tgt_ref, queue_ref, qlab_ref, out_ref, lab_ref, *, nfb):
    j = pl.program_id(0)

    @pl.when(j < nfb)
    def _insert():
        out_ref[...] = feats_ref[...].T
        lab_ref[...] = tgt_ref[...]

    @pl.when(j >= nfb)
    def _copy():
        out_ref[...] = queue_ref[...]
        lab_ref[...] = qlab_ref[...]


def kernel(feats, targets, queue, queue_label):
    bsz, dim = feats.shape
    k = queue.shape[1]
    nfb = bsz // _BW
    nblocks = k // _BW
    targets2d = targets.reshape(1, bsz)

    body = functools.partial(_body, nfb=nfb)

    queue_new, label_new = pl.pallas_call(
        body,
        grid=(nblocks,),
        in_specs=[
            pl.BlockSpec((_BW, dim), lambda j: (jnp.minimum(j, nfb - 1), 0)),
            pl.BlockSpec((1, _BW), lambda j: (0, jnp.minimum(j, nfb - 1))),
            pl.BlockSpec((dim, _BW), lambda j: (0, jnp.maximum(j, nfb))),
            pl.BlockSpec((1, _BW), lambda j: (0, jnp.maximum(j, nfb))),
        ],
        out_specs=[
            pl.BlockSpec((dim, _BW), lambda j: (0, j)),
            pl.BlockSpec((1, _BW), lambda j: (0, j)),
        ],
        out_shape=[
            jax.ShapeDtypeStruct((dim, k), queue.dtype),
            jax.ShapeDtypeStruct((1, k), queue_label.dtype),
        ],
    )(feats, targets2d, queue, queue_label)

    new_ptr = jnp.full((1,), (0 + bsz) % k, dtype=jnp.int32)
    return queue_new, label_new, new_ptr
